# baseline (speedup 1.0000x reference)
import functools

import jax
import jax.numpy as jnp
from jax.experimental import pallas as pl
from jax.experimental.pallas import tpu as pltpu

_ROTS = ((13, 15, 26, 6), (17, 29, 16, 24))


def _gumbel_bits(lin):
    k0 = jnp.uint32(0)
    k1 = jnp.uint32(42)
    k2 = k0 ^ k1 ^ jnp.uint32(0x1BD11BDA)
    ks = (k0, k1, k2)
    x0 = jnp.zeros_like(lin)
    x1 = lin + k1
    for i in range(5):
        for r in _ROTS[i % 2]:
            x0 = x0 + x1
            x1 = (x1 << jnp.uint32(r)) | (x1 >> jnp.uint32(32 - r))
            x1 = x0 ^ x1
        x0 = x0 + ks[(i + 1) % 3]
        x1 = x1 + ks[(i + 2) % 3] + jnp.uint32(i + 1)
    return x0 ^ x1


def _body(x_ref, samples_ref, lp_ref, acc_m, acc_i, acc_x, acc_s, *, cb, v):
    i = pl.program_id(0)
    g = pl.num_programs(0)

    @pl.when(i == 0)
    def _init():
        acc_m[...] = jnp.full_like(acc_m, -jnp.inf)
        acc_i[...] = jnp.zeros_like(acc_i)
        acc_x[...] = jnp.zeros_like(acc_x)
        acc_s[...] = jnp.zeros_like(acc_s)

    x = x_ref[...]
    b, _ = x.shape
    col = jax.lax.broadcasted_iota(jnp.int32, (b, cb), 1) + i * cb
    roff = jax.lax.broadcasted_iota(jnp.int32, (b, 1), 0) * v
    lin = (roff + col).astype(jnp.uint32)

    bits = _gumbel_bits(lin)
    u = jax.lax.bitcast_convert_type(
        (bits >> jnp.uint32(9)) | jnp.uint32(0x3F800000), jnp.float32) - 1.0
    gum = -jnp.log(-jnp.log(u))

    valid = col < v
    y = jnp.where(valid, x + gum, -jnp.inf)

    bm = jnp.max(y, axis=1, keepdims=True)
    at_max = y == bm
    bidx = jnp.min(jnp.where(at_max, col, jnp.int32(0x7FFFFFFF)),
                   axis=1, keepdims=True)
    bx = jnp.sum(jnp.where(col == bidx, x, 0.0), axis=1, keepdims=True)
    bs = jnp.sum(jnp.where(valid, jnp.exp(x), 0.0), axis=1, keepdims=True)

    upd = bm > acc_m[...]
    acc_i[...] = jnp.where(upd, bidx, acc_i[...])
    acc_x[...] = jnp.where(upd, bx, acc_x[...])
    acc_m[...] = jnp.where(upd, bm, acc_m[...])
    acc_s[...] = acc_s[...] + bs

    @pl.when(i == g - 1)
    def _finish():
        samples_ref[...] = acc_i[...]
        lp_ref[...] = acc_x[...] - jnp.log(acc_s[...])


def kernel(logits):
    b, v = logits.shape
    cb = 1024
    grid = (pl.cdiv(v, cb),)
    samples, lp = pl.pallas_call(
        functools.partial(_body, cb=cb, v=v),
        grid=grid,
        in_specs=[pl.BlockSpec((b, cb), lambda i: (0, i))],
        out_specs=[pl.BlockSpec((b, 1), lambda i: (0, 0)),
                   pl.BlockSpec((b, 1), lambda i: (0, 0))],
        out_shape=[jax.ShapeDtypeStruct((b, 1), jnp.int32),
                   jax.ShapeDtypeStruct((b, 1), jnp.float32)],
        scratch_shapes=[
            pltpu.VMEM((b, 1), jnp.float32),
            pltpu.VMEM((b, 1), jnp.int32),
            pltpu.VMEM((b, 1), jnp.float32),
            pltpu.VMEM((b, 1), jnp.float32),
        ],
    )(logits)
    return samples[:, 0], lp[:, 0]


# speedup vs baseline: 1.0611x; 1.0611x over previous
import functools

import jax
import jax.numpy as jnp
from jax.experimental import pallas as pl
from jax.experimental.pallas import tpu as pltpu

_ROTS = ((13, 15, 26, 6), (17, 29, 16, 24))


def _gumbel_bits(lin):
    k0 = jnp.uint32(0)
    k1 = jnp.uint32(42)
    k2 = k0 ^ k1 ^ jnp.uint32(0x1BD11BDA)
    ks = (k0, k1, k2)
    x0 = jnp.zeros_like(lin)
    x1 = lin + k1
    for i in range(5):
        for r in _ROTS[i % 2]:
            x0 = x0 + x1
            x1 = (x1 << jnp.uint32(r)) | (x1 >> jnp.uint32(32 - r))
            x1 = x0 ^ x1
        x0 = x0 + ks[(i + 1) % 3]
        x1 = x1 + ks[(i + 2) % 3] + jnp.uint32(i + 1)
    return x0 ^ x1


def _body(x_ref, samples_ref, lp_ref, acc_m, acc_i, acc_s, *, cb, ck, v):
    i = pl.program_id(0)
    g = pl.num_programs(0)

    @pl.when(i == 0)
    def _init():
        acc_m[...] = jnp.full_like(acc_m, -jnp.inf)
        acc_i[...] = jnp.zeros_like(acc_i)
        acc_s[...] = jnp.zeros_like(acc_s)

    b = x_ref.shape[0]
    roff = jax.lax.broadcasted_iota(jnp.int32, (b, 1), 0) * v

    def chunk(j, carry):
        cm, ci, cs = carry
        x = x_ref[:, pl.ds(j * ck, ck)]
        col = jax.lax.broadcasted_iota(jnp.int32, (b, ck), 1) + (i * cb + j * ck)
        lin = (roff + col).astype(jnp.uint32)

        bits = _gumbel_bits(lin)
        u = jax.lax.bitcast_convert_type(
            (bits >> jnp.uint32(9)) | jnp.uint32(0x3F800000), jnp.float32) - 1.0
        gum = -jnp.log(-jnp.log(u))

        valid = col < v
        y = jnp.where(valid, x + gum, -jnp.inf)

        bm = jnp.max(y, axis=1, keepdims=True)
        bidx = jnp.min(jnp.where(y == bm, col, jnp.int32(0x7FFFFFFF)),
                       axis=1, keepdims=True)
        bs = jnp.sum(jnp.where(valid, jnp.exp(x), 0.0), axis=1, keepdims=True)

        upd = bm > cm
        return (jnp.where(upd, bm, cm), jnp.where(upd, bidx, ci), cs + bs)

    cm, ci, cs = jax.lax.fori_loop(
        0, cb // ck,
        chunk,
        (jnp.full((b, 1), -jnp.inf, jnp.float32),
         jnp.zeros((b, 1), jnp.int32),
         jnp.zeros((b, 1), jnp.float32)),
    )

    upd = cm > acc_m[...]
    acc_i[...] = jnp.where(upd, ci, acc_i[...])
    acc_m[...] = jnp.where(upd, cm, acc_m[...])
    acc_s[...] = acc_s[...] + cs

    @pl.when(i == g - 1)
    def _finish():
        idx = acc_i[...]
        lin = (roff + idx).astype(jnp.uint32)
        bits = _gumbel_bits(lin)
        u = jax.lax.bitcast_convert_type(
            (bits >> jnp.uint32(9)) | jnp.uint32(0x3F800000), jnp.float32) - 1.0
        gum = -jnp.log(-jnp.log(u))
        samples_ref[...] = idx
        lp_ref[...] = (acc_m[...] - gum) - jnp.log(acc_s[...])


def kernel(logits):
    b, v = logits.shape
    cb = 8192
    ck = 1024
    grid = (pl.cdiv(v, cb),)
    samples, lp = pl.pallas_call(
        functools.partial(_body, cb=cb, ck=ck, v=v),
        grid=grid,
        in_specs=[pl.BlockSpec((b, cb), lambda i: (0, i))],
        out_specs=[pl.BlockSpec((b, 1), lambda i: (0, 0)),
                   pl.BlockSpec((b, 1), lambda i: (0, 0))],
        out_shape=[jax.ShapeDtypeStruct((b, 1), jnp.int32),
                   jax.ShapeDtypeStruct((b, 1), jnp.float32)],
        scratch_shapes=[
            pltpu.VMEM((b, 1), jnp.float32),
            pltpu.VMEM((b, 1), jnp.int32),
            pltpu.VMEM((b, 1), jnp.float32),
        ],
    )(logits)
    return samples[:, 0], lp[:, 0]


# speedup vs baseline: 1.1253x; 1.0605x over previous
import functools

import jax
import jax.numpy as jnp
from jax.experimental import pallas as pl
from jax.experimental.pallas import tpu as pltpu

_ROTS = ((13, 15, 26, 6), (17, 29, 16, 24))


def _gumbel_bits(lin):
    k0 = jnp.uint32(0)
    k1 = jnp.uint32(42)
    k2 = k0 ^ k1 ^ jnp.uint32(0x1BD11BDA)
    ks = (k0, k1, k2)
    x0 = jnp.zeros_like(lin)
    x1 = lin + k1
    for i in range(5):
        for r in _ROTS[i % 2]:
            x0 = x0 + x1
            x1 = (x1 << jnp.uint32(r)) | (x1 >> jnp.uint32(32 - r))
            x1 = x0 ^ x1
        x0 = x0 + ks[(i + 1) % 3]
        x1 = x1 + ks[(i + 2) % 3] + jnp.uint32(i + 1)
    return x0 ^ x1


def _body(x_ref, samples_ref, lp_ref, acc_m, acc_i, acc_s, *, cb, ck, v):
    i = pl.program_id(0)
    g = pl.num_programs(0)

    @pl.when(i == 0)
    def _init():
        acc_m[...] = jnp.full_like(acc_m, -jnp.inf)
        acc_i[...] = jnp.zeros_like(acc_i)
        acc_s[...] = jnp.zeros_like(acc_s)

    b = x_ref.shape[0]
    roff = jax.lax.broadcasted_iota(jnp.int32, (b, 1), 0) * v

    def chunk(j, carry):
        cm, ci, cs = carry
        x = x_ref[:, pl.ds(j * ck, ck)]
        col = jax.lax.broadcasted_iota(jnp.int32, (b, ck), 1) + (i * cb + j * ck)
        lin = (roff + col).astype(jnp.uint32)

        bits = _gumbel_bits(lin)
        u = jax.lax.bitcast_convert_type(
            (bits >> jnp.uint32(9)) | jnp.uint32(0x3F800000), jnp.float32) - 1.0
        gum = -jnp.log(-jnp.log(u))

        valid = col < v
        y = jnp.where(valid, x + gum, -jnp.inf)

        bm = jnp.max(y, axis=1, keepdims=True)
        bidx = jnp.min(jnp.where(y == bm, col, jnp.int32(0x7FFFFFFF)),
                       axis=1, keepdims=True)
        bs = jnp.sum(jnp.where(valid, jnp.exp(x), 0.0), axis=1, keepdims=True)

        upd = bm > cm
        return (jnp.where(upd, bm, cm), jnp.where(upd, bidx, ci), cs + bs)

    cm, ci, cs = jax.lax.fori_loop(
        0, cb // ck,
        chunk,
        (jnp.full((b, 1), -jnp.inf, jnp.float32),
         jnp.zeros((b, 1), jnp.int32),
         jnp.zeros((b, 1), jnp.float32)),
    )

    upd = cm > acc_m[...]
    acc_i[...] = jnp.where(upd, ci, acc_i[...])
    acc_m[...] = jnp.where(upd, cm, acc_m[...])
    acc_s[...] = acc_s[...] + cs

    @pl.when(i == g - 1)
    def _finish():
        idx = acc_i[...]
        lin = (roff + idx).astype(jnp.uint32)
        bits = _gumbel_bits(lin)
        u = jax.lax.bitcast_convert_type(
            (bits >> jnp.uint32(9)) | jnp.uint32(0x3F800000), jnp.float32) - 1.0
        gum = -jnp.log(-jnp.log(u))
        samples_ref[...] = idx
        lp_ref[...] = (acc_m[...] - gum) - jnp.log(acc_s[...])


def kernel(logits):
    b, v = logits.shape
    cb = 8192
    ck = 2048
    grid = (pl.cdiv(v, cb),)
    samples, lp = pl.pallas_call(
        functools.partial(_body, cb=cb, ck=ck, v=v),
        grid=grid,
        in_specs=[pl.BlockSpec((b, cb), lambda i: (0, i))],
        out_specs=[pl.BlockSpec((b, 1), lambda i: (0, 0)),
                   pl.BlockSpec((b, 1), lambda i: (0, 0))],
        out_shape=[jax.ShapeDtypeStruct((b, 1), jnp.int32),
                   jax.ShapeDtypeStruct((b, 1), jnp.float32)],
        scratch_shapes=[
            pltpu.VMEM((b, 1), jnp.float32),
            pltpu.VMEM((b, 1), jnp.int32),
            pltpu.VMEM((b, 1), jnp.float32),
        ],
    )(logits)
    return samples[:, 0], lp[:, 0]


# speedup vs baseline: 1.2263x; 1.0898x over previous
import functools

import jax
import jax.numpy as jnp
from jax.experimental import pallas as pl
from jax.experimental.pallas import tpu as pltpu

_ROTS = ((13, 15, 26, 6), (17, 29, 16, 24))


def _gumbel_bits(lin):
    k0 = jnp.uint32(0)
    k1 = jnp.uint32(42)
    k2 = k0 ^ k1 ^ jnp.uint32(0x1BD11BDA)
    ks = (k0, k1, k2)
    x0 = jnp.zeros_like(lin)
    x1 = lin + k1
    for i in range(5):
        for r in _ROTS[i % 2]:
            x0 = x0 + x1
            x1 = (x1 << jnp.uint32(r)) | (x1 >> jnp.uint32(32 - r))
            x1 = x0 ^ x1
        x0 = x0 + ks[(i + 1) % 3]
        x1 = x1 + ks[(i + 2) % 3] + jnp.uint32(i + 1)
    return x0 ^ x1


def _gumbel(lin):
    bits = _gumbel_bits(lin)
    u = jax.lax.bitcast_convert_type(
        (bits >> jnp.uint32(9)) | jnp.uint32(0x3F800000), jnp.float32) - 1.0
    return -jnp.log(-jnp.log(u))


def _body(x_ref, samples_ref, lp_ref, acc_y, acc_i, acc_s, *, cb, v):
    i = pl.program_id(0)
    g = pl.num_programs(0)
    b = x_ref.shape[0]

    @pl.when(i == 0)
    def _init():
        acc_y[...] = jnp.full_like(acc_y, -jnp.inf)
        acc_i[...] = jnp.zeros_like(acc_i)
        acc_s[...] = jnp.zeros_like(acc_s)

    roff = jax.lax.broadcasted_iota(jnp.int32, (b, 1), 0) * v
    lane = jax.lax.broadcasted_iota(jnp.int32, (b, 128), 1)
    base_lin = roff + lane

    ay = acc_y[...]
    ai = acc_i[...]
    asum = acc_s[...]
    for j in range(cb // 128):
        x = x_ref[:, j * 128:(j + 1) * 128]
        off = i * cb + j * 128
        lin = base_lin + off
        gum = _gumbel(lin.astype(jnp.uint32))
        valid = lane < (v - off)
        y = jnp.where(valid, x + gum, -jnp.inf)
        upd = y > ay
        ay = jnp.where(upd, y, ay)
        ai = jnp.where(upd, lin, ai)
        asum = asum + jnp.where(valid, jnp.exp(x), 0.0)
    acc_y[...] = ay
    acc_i[...] = ai
    acc_s[...] = asum

    @pl.when(i == g - 1)
    def _finish():
        ayf = acc_y[...]
        m = jnp.max(ayf, axis=1, keepdims=True)
        lin_w = jnp.min(
            jnp.where(ayf == m, acc_i[...], jnp.int32(0x7FFFFFFF)),
            axis=1, keepdims=True)
        gum_w = _gumbel(lin_w.astype(jnp.uint32))
        lse = jnp.log(jnp.sum(acc_s[...], axis=1, keepdims=True))
        samples_ref[...] = lin_w - roff
        lp_ref[...] = (m - gum_w) - lse


def kernel(logits):
    b, v = logits.shape
    cb = 2048
    grid = (pl.cdiv(v, cb),)
    samples, lp = pl.pallas_call(
        functools.partial(_body, cb=cb, v=v),
        grid=grid,
        in_specs=[pl.BlockSpec((b, cb), lambda i: (0, i))],
        out_specs=[pl.BlockSpec((b, 1), lambda i: (0, 0)),
                   pl.BlockSpec((b, 1), lambda i: (0, 0))],
        out_shape=[jax.ShapeDtypeStruct((b, 1), jnp.int32),
                   jax.ShapeDtypeStruct((b, 1), jnp.float32)],
        scratch_shapes=[
            pltpu.VMEM((b, 128), jnp.float32),
            pltpu.VMEM((b, 128), jnp.int32),
            pltpu.VMEM((b, 128), jnp.float32),
        ],
    )(logits)
    return samples[:, 0], lp[:, 0]
